# Initial kernel scaffold; baseline (speedup 1.0000x reference)
import jax
import jax.numpy as jnp
from jax.experimental import pallas as pl


def kernel(input_ids, dataset_embeddings, W1, b1, W2, b2):
    raise NotImplementedError("write your pallas kernel here")


# speedup vs baseline: 1.1209x; 1.1209x over previous
import functools

import jax
import jax.numpy as jnp
from jax import lax
from jax.experimental import pallas as pl
from jax.experimental.pallas import tpu as pltpu
from jax.experimental.pallas import tpu_sc as plsc

H = 1024
NSP = 8
CORPUS = 512
ROWS = CORPUS + NSP
BATCH = 32
W2_BLOCKS = NSP


def _mlp_body(w1_ref, b1_ref, w2_ref, b2_ref, sp_ref, h_ref):
    r = pl.program_id(0)

    @pl.when(r == 0)
    def _():
        ones = jnp.ones((1, H), jnp.float32)
        h = lax.dot_general(ones, w1_ref[...], (((1,), (1,)), ((), ())),
                            preferred_element_type=jnp.float32)
        h_ref[...] = jax.nn.relu(h + b1_ref[...])

    sp_ref[0] = lax.dot_general(h_ref[...], w2_ref[...],
                                (((1,), (1,)), ((), ())),
                                preferred_element_type=jnp.float32) + b2_ref[0]


def _soft_prompt(W1, b1, W2, b2):
    b1r = b1.reshape(1, H)
    b2r = b2.reshape(NSP, 1, H)
    sp = pl.pallas_call(
        _mlp_body,
        grid=(W2_BLOCKS,),
        in_specs=[
            pl.BlockSpec((H, H), lambda r: (0, 0)),
            pl.BlockSpec((1, H), lambda r: (0, 0)),
            pl.BlockSpec((H, H), lambda r: (r, 0)),
            pl.BlockSpec((1, 1, H), lambda r: (r, 0, 0)),
        ],
        out_specs=pl.BlockSpec((1, 1, H), lambda r: (r, 0, 0)),
        out_shape=jax.ShapeDtypeStruct((NSP, 1, H), jnp.float32),
        scratch_shapes=[pltpu.VMEM((1, H), jnp.float32)],
    )(W1, b1r, W2, b2r)
    return sp.reshape(NSP, H)


def _sc_broadcast(de, sp):
    mesh = plsc.VectorSubcoreMesh(core_axis_name="c", subcore_axis_name="s")

    @functools.partial(
        pl.kernel,
        out_type=jax.ShapeDtypeStruct((BATCH, ROWS, H), jnp.float32),
        mesh=mesh,
        scratch_types=[pltpu.VMEM_SHARED((ROWS, H), jnp.float32)],
    )
    def body(de_hbm, sp_hbm, out_hbm, shared):
        c = lax.axis_index("c")
        s = lax.axis_index("s")

        @pl.when(s == 0)
        def _():
            pltpu.sync_copy(de_hbm, shared.at[pl.ds(0, CORPUS)])

        @pl.when(s == 1)
        def _():
            pltpu.sync_copy(sp_hbm, shared.at[pl.ds(CORPUS, NSP)])

        plsc.subcore_barrier()
        b = s * 2 + c
        pltpu.sync_copy(shared, out_hbm.at[b])

    return body(de, sp)


def kernel(input_ids, dataset_embeddings, W1, b1, W2, b2):
    del input_ids
    sp = _soft_prompt(W1, b1, W2, b2)
    return _sc_broadcast(dataset_embeddings.astype(jnp.float32), sp)


# speedup vs baseline: 1.3060x; 1.1651x over previous
import functools

import jax
import jax.numpy as jnp
from jax import lax
from jax.experimental import pallas as pl
from jax.experimental.pallas import tpu as pltpu
from jax.experimental.pallas import tpu_sc as plsc

H = 1024
NSP = 8
CORPUS = 512
ROWS = CORPUS + NSP
BATCH = 32
W2_BLOCKS = NSP


def _mlp_body(w1_ref, b1_ref, w2_ref, b2_ref, sp_ref, h_ref):
    r = pl.program_id(0)

    @pl.when(r == 0)
    def _():
        ones = jnp.ones((1, H), jnp.float32)
        h = lax.dot_general(ones, w1_ref[...], (((1,), (1,)), ((), ())),
                            preferred_element_type=jnp.float32)
        h_ref[...] = jax.nn.relu(h + b1_ref[...])

    sp_ref[0] = lax.dot_general(h_ref[...], w2_ref[...],
                                (((1,), (1,)), ((), ())),
                                preferred_element_type=jnp.float32) + b2_ref[0]


def _soft_prompt(W1, b1, W2, b2):
    b1r = b1.reshape(1, H)
    b2r = b2.reshape(NSP, 1, H)
    sp = pl.pallas_call(
        _mlp_body,
        grid=(W2_BLOCKS,),
        in_specs=[
            pl.BlockSpec((H, H), lambda r: (0, 0)),
            pl.BlockSpec((1, H), lambda r: (0, 0)),
            pl.BlockSpec((H, H), lambda r: (r, 0)),
            pl.BlockSpec((1, 1, H), lambda r: (r, 0, 0)),
        ],
        out_specs=pl.BlockSpec((1, 1, H), lambda r: (r, 0, 0)),
        out_shape=jax.ShapeDtypeStruct((NSP, 1, H), jnp.float32),
        scratch_shapes=[pltpu.VMEM((1, H), jnp.float32)],
    )(W1, b1r, W2, b2r)
    return sp.reshape(NSP, H)


def _sc_broadcast_de(de):
    mesh = plsc.VectorSubcoreMesh(core_axis_name="c", subcore_axis_name="s")

    @functools.partial(
        pl.kernel,
        out_type=jax.ShapeDtypeStruct((BATCH, ROWS, H), jnp.float32),
        mesh=mesh,
        scratch_types=[pltpu.VMEM_SHARED((CORPUS, H), jnp.float32)],
    )
    def body(de_hbm, out_hbm, shared):
        c = lax.axis_index("c")
        s = lax.axis_index("s")

        @pl.when(s == 0)
        def _():
            pltpu.sync_copy(de_hbm, shared)

        plsc.subcore_barrier()
        b = s * 2 + c
        pltpu.sync_copy(shared, out_hbm.at[b, pl.ds(0, CORPUS)])

    return body(de)


def _sp_write_body(out_alias_ref, sp_ref, out_ref):
    del out_alias_ref
    out_ref[...] = jnp.broadcast_to(sp_ref[...][None], (BATCH, NSP, H))


def _sp_write(out1, sp):
    return pl.pallas_call(
        _sp_write_body,
        grid=(1,),
        in_specs=[
            pl.BlockSpec(memory_space=pl.ANY),
            pl.BlockSpec((NSP, H), lambda i: (0, 0)),
        ],
        out_specs=pl.BlockSpec((BATCH, NSP, H), lambda i: (0, 64, 0)),
        out_shape=jax.ShapeDtypeStruct((BATCH, ROWS, H), jnp.float32),
        input_output_aliases={0: 0},
    )(out1, sp)


def kernel(input_ids, dataset_embeddings, W1, b1, W2, b2):
    del input_ids
    sp = _soft_prompt(W1, b1, W2, b2)
    out1 = _sc_broadcast_de(dataset_embeddings.astype(jnp.float32))
    return _sp_write(out1, sp)


# speedup vs baseline: 1.3392x; 1.0254x over previous
import functools

import jax
import jax.numpy as jnp
from jax import lax
from jax.experimental import pallas as pl
from jax.experimental.pallas import tpu as pltpu
from jax.experimental.pallas import tpu_sc as plsc

H = 1024
NSP = 8
CORPUS = 512
ROWS = CORPUS + NSP
BATCH = 32
SC_BATCHES = 16
W2_ROWS = NSP * H
W2_BLK = 2048
N_WORKERS = 32


def _mlp_body(w1_ref, b1_ref, w2_ref, b2_ref, sp_ref, h_ref):
    r = pl.program_id(0)

    @pl.when(r == 0)
    def _():
        ones = jnp.ones((8, H), jnp.float32)
        h = lax.dot_general(ones, w1_ref[...], (((1,), (1,)), ((), ())),
                            preferred_element_type=jnp.float32)
        h_ref[...] = jax.nn.relu(h + b1_ref[...])

    res = lax.dot_general(h_ref[...], w2_ref[...], (((1,), (1,)), ((), ())),
                          preferred_element_type=jnp.float32)
    sp_ref[...] = res[0:1, :] + b2_ref[...]


def _soft_prompt(W1, b1, W2, b2):
    b1r = b1.reshape(1, H)
    b2r = b2.reshape(1, W2_ROWS)
    sp = pl.pallas_call(
        _mlp_body,
        grid=(W2_ROWS // W2_BLK,),
        in_specs=[
            pl.BlockSpec((H, H), lambda r: (0, 0)),
            pl.BlockSpec((1, H), lambda r: (0, 0)),
            pl.BlockSpec((W2_BLK, H), lambda r: (r, 0)),
            pl.BlockSpec((1, W2_BLK), lambda r: (0, r)),
        ],
        out_specs=pl.BlockSpec((1, W2_BLK), lambda r: (0, r)),
        out_shape=jax.ShapeDtypeStruct((1, W2_ROWS), jnp.float32),
        scratch_shapes=[pltpu.VMEM((8, H), jnp.float32)],
    )(W1, b1r, W2, b2r)
    return sp.reshape(NSP, H)


def _sc_broadcast_de(de):
    rows_per_w = SC_BATCHES * CORPUS // N_WORKERS
    mesh = plsc.VectorSubcoreMesh(core_axis_name="c", subcore_axis_name="s")

    @functools.partial(
        pl.kernel,
        out_type=jax.ShapeDtypeStruct((BATCH, ROWS, H), jnp.float32),
        mesh=mesh,
        scratch_types=[pltpu.VMEM_SHARED((CORPUS, H), jnp.float32)],
    )
    def body(de_hbm, out_hbm, shared):
        c = lax.axis_index("c")
        s = lax.axis_index("s")

        @pl.when(s == 0)
        def _():
            pltpu.sync_copy(de_hbm, shared)

        plsc.subcore_barrier()
        w = s * 2 + c
        flat = w * rows_per_w
        b = flat // CORPUS
        r0 = flat % CORPUS
        pltpu.sync_copy(shared.at[pl.ds(r0, rows_per_w)],
                        out_hbm.at[b, pl.ds(r0, rows_per_w)])

    return body(de)


def _sp_write_body(out_alias_ref, sp_ref, out_ref):
    del out_alias_ref
    out_ref[...] = jnp.broadcast_to(sp_ref[...][None], (BATCH, NSP, H))


def _sp_write(out1, sp):
    return pl.pallas_call(
        _sp_write_body,
        grid=(1,),
        in_specs=[
            pl.BlockSpec(memory_space=pl.ANY),
            pl.BlockSpec((NSP, H), lambda i: (0, 0)),
        ],
        out_specs=pl.BlockSpec((BATCH, NSP, H), lambda i: (0, 64, 0)),
        out_shape=jax.ShapeDtypeStruct((BATCH, ROWS, H), jnp.float32),
        input_output_aliases={0: 0},
    )(out1, sp)


def _de_write_body(out_alias_ref, de_ref, out_ref):
    del out_alias_ref
    out_ref[...] = de_ref[...][None]


def _tc_de_write(out1, de):
    return pl.pallas_call(
        _de_write_body,
        grid=(BATCH - SC_BATCHES,),
        in_specs=[
            pl.BlockSpec(memory_space=pl.ANY),
            pl.BlockSpec((CORPUS, H), lambda b: (0, 0)),
        ],
        out_specs=pl.BlockSpec((1, CORPUS, H), lambda b: (b + SC_BATCHES, 0, 0)),
        out_shape=jax.ShapeDtypeStruct((BATCH, ROWS, H), jnp.float32),
        input_output_aliases={0: 0},
    )(out1, de)


def kernel(input_ids, dataset_embeddings, W1, b1, W2, b2):
    del input_ids
    de = dataset_embeddings.astype(jnp.float32)
    sp = _soft_prompt(W1, b1, W2, b2)
    out = _sc_broadcast_de(de)
    out = _tc_de_write(out, de)
    out = _sp_write(out, sp)
    return out
